# Initial kernel scaffold; baseline (speedup 1.0000x reference)
import functools

import jax
import jax.numpy as jnp
from jax import lax
from jax.experimental import pallas as pl
from jax.experimental.pallas import tpu as pltpu
from jax.experimental.pallas import tpu_sc as plsc

N = 10000
N_IN = 128
N_H = 64
N_IDX = 5000

_ROWS_PER_BLOCK = 1000
_GRID = N // _ROWS_PER_BLOCK

_NC = 2
_NS = 16
_NW = _NC * _NS
_LANES = 16
_PAD = ((N_IDX + 8 * _NW - 1) // (8 * _NW)) * (8 * _NW)
_CHUNK = _PAD // _NW


def _norm_body(x_ref, w1_ref, b1_ref, w2_ref, b2_ref, out_ref):
    x = x_ref[...]
    h = jnp.dot(x, w1_ref[...], preferred_element_type=jnp.float32) + b1_ref[...]
    h = jnp.maximum(h, 0.0)
    xr = jnp.dot(h, w2_ref[...], preferred_element_type=jnp.float32) + b2_ref[...]
    d = x - xr
    out_ref[...] = jnp.sqrt(jnp.sum(d * d, axis=1))


def _row_norms(x, w1, b1, w2, b2):
    return pl.pallas_call(
        _norm_body,
        grid=(_GRID,),
        in_specs=[
            pl.BlockSpec((_ROWS_PER_BLOCK, N_IN), lambda i: (i, 0)),
            pl.BlockSpec((N_IN, N_H), lambda i: (0, 0)),
            pl.BlockSpec((1, N_H), lambda i: (0, 0)),
            pl.BlockSpec((N_H, N_IN), lambda i: (0, 0)),
            pl.BlockSpec((1, N_IN), lambda i: (0, 0)),
        ],
        out_specs=pl.BlockSpec((_ROWS_PER_BLOCK,), lambda i: (i,)),
        out_shape=jax.ShapeDtypeStruct((N,), jnp.float32),
    )(x, w1, b1.reshape(1, N_H), w2, b2.reshape(1, N_IN))


def _sc_body(nrm_hbm, idx_tr_hbm, idx_te_hbm, te_out, part_out,
             idx_v, val_v, acc_v, sem):
    wid = lax.axis_index("s") * _NC + lax.axis_index("c")
    base = wid * _CHUNK
    pltpu.sync_copy(idx_te_hbm.at[pl.ds(base, _CHUNK)], idx_v)
    pltpu.async_copy(nrm_hbm.at[idx_v], val_v, sem).wait()
    pltpu.sync_copy(val_v, te_out.at[pl.ds(base, _CHUNK)])
    pltpu.sync_copy(idx_tr_hbm.at[pl.ds(base, _CHUNK)], idx_v)
    pltpu.async_copy(nrm_hbm.at[idx_v], val_v, sem).wait()
    lanes = lax.iota(jnp.int32, _LANES)
    acc = jnp.zeros((_LANES,), jnp.float32)
    for j in range(_CHUNK // _LANES):
        v = val_v[pl.ds(j * _LANES, _LANES)]
        g = lanes + (base + j * _LANES)
        acc = acc + jnp.where(g < N_IDX, v, 0.0)
    acc_v[...] = acc
    pltpu.sync_copy(acc_v, part_out.at[wid])


def _sc_gather(nrm, idx_tr, idx_te):
    mesh = plsc.VectorSubcoreMesh(core_axis_name="c", subcore_axis_name="s")
    run = functools.partial(
        pl.kernel,
        mesh=mesh,
        out_type=[
            jax.ShapeDtypeStruct((_PAD,), jnp.float32),
            jax.ShapeDtypeStruct((_NW, _LANES), jnp.float32),
        ],
        scratch_types=[
            pltpu.VMEM((_CHUNK,), jnp.int32),
            pltpu.VMEM((_CHUNK,), jnp.float32),
            pltpu.VMEM((_LANES,), jnp.float32),
            pltpu.SemaphoreType.DMA,
        ],
    )(_sc_body)
    return run(nrm, idx_tr, idx_te)


def kernel(seq1, adj, idx_train, idx_test, W_stru, b_stru,
           W_attr1, b_attr1, W_attr2, b_attr2):
    del adj, W_stru, b_stru
    nrm = _row_norms(seq1, W_attr1, b_attr1, W_attr2, b_attr2)
    idx_tr = jnp.pad(idx_train.astype(jnp.int32), (0, _PAD - N_IDX))
    idx_te = jnp.pad(idx_test.astype(jnp.int32), (0, _PAD - N_IDX))
    te, parts = _sc_gather(nrm, idx_tr, idx_te)
    loss = jnp.sum(parts) * (1.0 / N_IDX)
    return (loss, te[:N_IDX])


# speedup vs baseline: 1.6229x; 1.6229x over previous
import functools

import jax
import jax.numpy as jnp
from jax import lax
from jax.experimental import pallas as pl
from jax.experimental.pallas import tpu as pltpu
from jax.experimental.pallas import tpu_sc as plsc

N = 10000
N_IN = 128
N_H = 64
N_IDX = 5000

_ROWS_PER_BLOCK = 1000
_GRID = N // _ROWS_PER_BLOCK

_NC = 2
_NS = 16
_NW = _NC * _NS
_LANES = 16
_PAD = ((N_IDX + 8 * _NW - 1) // (8 * _NW)) * (8 * _NW)
_CHUNK = _PAD // _NW


def _norm_body(x_ref, w1_ref, b1_ref, w2_ref, b2_ref, out_ref):
    x = x_ref[...]
    h = jnp.dot(x, w1_ref[...], preferred_element_type=jnp.float32) + b1_ref[...]
    h = jnp.maximum(h, 0.0)
    xr = jnp.dot(h, w2_ref[...], preferred_element_type=jnp.float32) + b2_ref[...]
    d = x - xr
    out_ref[...] = jnp.sqrt(jnp.sum(d * d, axis=1))


def _row_norms(x, w1, b1, w2, b2):
    return pl.pallas_call(
        _norm_body,
        out_shape=jax.ShapeDtypeStruct((N,), jnp.float32),
    )(x, w1, b1.reshape(1, N_H), w2, b2.reshape(1, N_IN))


def _sc_body(nrm_hbm, idx_tr_hbm, idx_te_hbm, te_out, part_out,
             idx_v, val_v, acc_v, sem):
    wid = lax.axis_index("s") * _NC + lax.axis_index("c")
    base = wid * _CHUNK
    pltpu.sync_copy(idx_te_hbm.at[pl.ds(base, _CHUNK)], idx_v)
    pltpu.async_copy(nrm_hbm.at[idx_v], val_v, sem).wait()
    pltpu.sync_copy(val_v, te_out.at[pl.ds(base, _CHUNK)])
    pltpu.sync_copy(idx_tr_hbm.at[pl.ds(base, _CHUNK)], idx_v)
    pltpu.async_copy(nrm_hbm.at[idx_v], val_v, sem).wait()
    lanes = lax.iota(jnp.int32, _LANES)
    acc = jnp.zeros((_LANES,), jnp.float32)
    for j in range(_CHUNK // _LANES):
        v = val_v[pl.ds(j * _LANES, _LANES)]
        g = lanes + (base + j * _LANES)
        acc = acc + jnp.where(g < N_IDX, v, 0.0)
    acc_v[...] = acc
    pltpu.sync_copy(acc_v, part_out.at[wid])


def _sc_gather(nrm, idx_tr, idx_te):
    mesh = plsc.VectorSubcoreMesh(core_axis_name="c", subcore_axis_name="s")
    run = functools.partial(
        pl.kernel,
        mesh=mesh,
        out_type=[
            jax.ShapeDtypeStruct((_PAD,), jnp.float32),
            jax.ShapeDtypeStruct((_NW, _LANES), jnp.float32),
        ],
        scratch_types=[
            pltpu.VMEM((_CHUNK,), jnp.int32),
            pltpu.VMEM((_CHUNK,), jnp.float32),
            pltpu.VMEM((_LANES,), jnp.float32),
            pltpu.SemaphoreType.DMA,
        ],
    )(_sc_body)
    return run(nrm, idx_tr, idx_te)


def kernel(seq1, adj, idx_train, idx_test, W_stru, b_stru,
           W_attr1, b_attr1, W_attr2, b_attr2):
    del adj, W_stru, b_stru
    nrm = _row_norms(seq1, W_attr1, b_attr1, W_attr2, b_attr2)
    idx_tr = jnp.pad(idx_train.astype(jnp.int32), (0, _PAD - N_IDX))
    idx_te = jnp.pad(idx_test.astype(jnp.int32), (0, _PAD - N_IDX))
    te, parts = _sc_gather(nrm, idx_tr, idx_te)
    loss = jnp.sum(parts) * (1.0 / N_IDX)
    return (loss, te[:N_IDX])
